# Initial kernel scaffold; baseline (speedup 1.0000x reference)
import jax
import jax.numpy as jnp
from jax.experimental import pallas as pl


def kernel(x, table):
    raise NotImplementedError("write your pallas kernel here")


# speedup vs baseline: 2.4144x; 2.4144x over previous
import functools

import jax
import jax.numpy as jnp
from jax import lax
from jax.experimental import pallas as pl
from jax.experimental.pallas import tpu as pltpu
from jax.experimental.pallas import tpu_sc as plsc

D_MODEL = 128
SCALE = float(D_MODEL) ** 0.5

_NC = 2
_NS = 16
_NW = _NC * _NS

_CHUNK = 128
_LANES = 16


def _make_kernel(n_tokens: int):
    assert n_tokens % (_NW * _CHUNK) == 0
    nch = n_tokens // (_NW * _CHUNK)

    mesh = plsc.VectorSubcoreMesh(core_axis_name="c", subcore_axis_name="s")

    @functools.partial(
        pl.kernel,
        out_type=jax.ShapeDtypeStruct((n_tokens, D_MODEL), jnp.float32),
        mesh=mesh,
        scratch_types=[
            pltpu.VMEM((nch * _CHUNK,), jnp.int32),
            pltpu.VMEM((_CHUNK, D_MODEL), jnp.float32),
            pltpu.SemaphoreType.DMA,
        ],
    )
    def emb_kernel(x_hbm, table_hbm, out_hbm, idx_v, rows_v, gsem):
        wid = lax.axis_index("s") * _NC + lax.axis_index("c")
        pltpu.sync_copy(x_hbm.at[pl.ds(wid * nch * _CHUNK, nch * _CHUNK)], idx_v)

        def chunk_body(j, carry):
            pltpu.async_copy(
                table_hbm.at[idx_v.at[pl.ds(j * _CHUNK, _CHUNK)]], rows_v, gsem
            ).wait()

            def scale_row(r, c2):
                for c in range(D_MODEL // _LANES):
                    sl = pl.ds(c * _LANES, _LANES)
                    rows_v[r, sl] = rows_v[r, sl] * SCALE
                return c2

            lax.fori_loop(0, _CHUNK, scale_row, 0)

            base = (wid * nch + j) * _CHUNK
            pltpu.sync_copy(rows_v, out_hbm.at[pl.ds(base, _CHUNK)])
            return carry

        lax.fori_loop(0, nch, chunk_body, 0)

    return emb_kernel


def kernel(x, table):
    b, s = x.shape
    n_tokens = b * s
    idx_flat = jnp.reshape(x.astype(jnp.int32), (n_tokens,))
    out = _make_kernel(n_tokens)(idx_flat, table)
    return jnp.reshape(out, (b, s, D_MODEL))


# speedup vs baseline: 2.9555x; 1.2241x over previous
import functools

import jax
import jax.numpy as jnp
from jax import lax
from jax.experimental import pallas as pl
from jax.experimental.pallas import tpu as pltpu
from jax.experimental.pallas import tpu_sc as plsc

D_MODEL = 128
SCALE = float(D_MODEL) ** 0.5

_NC = 2
_NS = 16
_NW = _NC * _NS

_CHUNK = 128
_LANES = 16
_NBUF = 5
_K = 3


def _make_kernel(n_tokens: int):
    assert n_tokens % (_NW * _CHUNK) == 0
    nch = n_tokens // (_NW * _CHUNK)
    assert nch % _NBUF == 0

    mesh = plsc.VectorSubcoreMesh(core_axis_name="c", subcore_axis_name="s")

    @functools.partial(
        pl.kernel,
        out_type=jax.ShapeDtypeStruct((n_tokens, D_MODEL), jnp.float32),
        mesh=mesh,
        scratch_types=(
            [pltpu.VMEM((nch * _CHUNK,), jnp.int32)]
            + [pltpu.VMEM((_CHUNK, D_MODEL), jnp.float32)] * _NBUF
            + [pltpu.SemaphoreType.DMA] * (2 * _NBUF)
        ),
    )
    def emb_kernel(x_hbm, table_hbm, out_hbm, idx_v, *bufs_and_sems):
        rows = bufs_and_sems[:_NBUF]
        gsem = bufs_and_sems[_NBUF:2 * _NBUF]
        ssem = bufs_and_sems[2 * _NBUF:]

        wid = lax.axis_index("s") * _NC + lax.axis_index("c")
        ibase = wid * nch * _CHUNK
        pltpu.sync_copy(x_hbm.at[pl.ds(ibase, nch * _CHUNK)], idx_v)

        def start_gather(j, b):
            pltpu.async_copy(
                table_hbm.at[idx_v.at[pl.ds(j * _CHUNK, _CHUNK)]],
                rows[b], gsem[b])

        def wait_gather(b):
            pltpu.make_async_copy(
                table_hbm.at[idx_v.at[pl.ds(0, _CHUNK)]],
                rows[b], gsem[b]).wait()

        def start_store(j, b):
            pltpu.async_copy(
                rows[b], out_hbm.at[pl.ds(ibase + j * _CHUNK, _CHUNK)],
                ssem[b])

        def wait_store(b):
            pltpu.make_async_copy(
                rows[b], out_hbm.at[pl.ds(ibase, _CHUNK)], ssem[b]).wait()

        for b in range(_K):
            start_gather(b, b)

        def outer(o, carry):
            for b in range(_NBUF):
                j = o * _NBUF + b
                jn = j + _K
                bn = (b + _K) % _NBUF

                @pl.when(jn < nch)
                def _():
                    @pl.when(j >= _NBUF - _K)
                    def _():
                        wait_store(bn)
                    start_gather(jn, bn)

                wait_gather(b)

                def scale_row(r, c2):
                    for c in range(D_MODEL // _LANES):
                        sl = pl.ds(c * _LANES, _LANES)
                        rows[b][r, sl] = rows[b][r, sl] * SCALE
                    return c2

                lax.fori_loop(0, _CHUNK, scale_row, 0)
                start_store(j, b)
            return carry

        lax.fori_loop(0, nch // _NBUF, outer, 0)

        for b in range(_NBUF):
            wait_store(b)

    return emb_kernel


def kernel(x, table):
    b, s = x.shape
    n_tokens = b * s
    idx_flat = jnp.reshape(x.astype(jnp.int32), (n_tokens,))
    out = _make_kernel(n_tokens)(idx_flat, table)
    return jnp.reshape(out, (b, s, D_MODEL))


# speedup vs baseline: 5.2797x; 1.7864x over previous
import functools

import jax
import jax.numpy as jnp
from jax import lax
from jax.experimental import pallas as pl
from jax.experimental.pallas import tpu as pltpu
from jax.experimental.pallas import tpu_sc as plsc

D_MODEL = 128
SCALE = float(D_MODEL) ** 0.5

_NC = 2
_NS = 16
_NW = _NC * _NS

_LANES = 16
_NBUF = 8
_K = 5


def _make_kernel(batch: int, seq: int):
    assert batch % _NW == 0
    nch = batch // _NW
    assert nch >= _NBUF

    mesh = plsc.VectorSubcoreMesh(core_axis_name="c", subcore_axis_name="s")

    @functools.partial(
        pl.kernel,
        out_type=jax.ShapeDtypeStruct((batch, seq, D_MODEL), jnp.float32),
        mesh=mesh,
        scratch_types=(
            [pltpu.VMEM((nch, seq), jnp.int32)]
            + [pltpu.VMEM((seq, D_MODEL), jnp.float32)] * _NBUF
            + [pltpu.SemaphoreType.DMA] * (2 * _NBUF)
        ),
    )
    def emb_kernel(x_hbm, table_hbm, out_hbm, idx_v, *bufs_and_sems):
        rows = bufs_and_sems[:_NBUF]
        gsem = bufs_and_sems[_NBUF:2 * _NBUF]
        ssem = bufs_and_sems[2 * _NBUF:]

        wid = lax.axis_index("s") * _NC + lax.axis_index("c")
        row0 = wid * nch
        pltpu.sync_copy(x_hbm.at[pl.ds(row0, nch)], idx_v)

        def start_gather(j, b):
            pltpu.async_copy(table_hbm.at[idx_v.at[j]], rows[b], gsem[b])

        def wait_gather(b):
            pltpu.make_async_copy(
                table_hbm.at[idx_v.at[0]], rows[b], gsem[b]).wait()

        def start_store(j, b):
            pltpu.async_copy(rows[b], out_hbm.at[row0 + j], ssem[b])

        def wait_store(b):
            pltpu.make_async_copy(
                rows[b], out_hbm.at[row0], ssem[b]).wait()

        for b in range(_K):
            start_gather(b, b)

        def _scale(b):
            def scale_row(r, c2):
                for c in range(D_MODEL // _LANES):
                    sl = pl.ds(c * _LANES, _LANES)
                    rows[b][r, sl] = rows[b][r, sl] * SCALE
                return c2
            lax.fori_loop(0, seq, scale_row, 0)

        def outer(o, carry):
            for b in range(_NBUF):
                j = o * _NBUF + b
                jn = j + _K
                bn = (b + _K) % _NBUF

                @pl.when(jn < nch)
                def _():
                    @pl.when(j >= _NBUF - _K)
                    def _():
                        wait_store(bn)
                    start_gather(jn, bn)

                wait_gather(b)
                _scale(b)
                start_store(j, b)
            return carry

        assert nch % _NBUF == 0
        lax.fori_loop(0, nch // _NBUF, outer, 0)

        for b in range(_NBUF):
            wait_store(b)

    return emb_kernel


def kernel(x, table):
    b, s = x.shape
    return _make_kernel(b, s)(x.astype(jnp.int32), table)
